# speedup vs baseline: 1.3705x; 1.3705x over previous
import functools

import jax
import jax.numpy as jnp
from jax import lax
from jax.experimental import pallas as pl
from jax.experimental.pallas import tpu as pltpu

_NUM_LABELS = 1000
_CLAMP = 4.6051
_B = 16384
_D = 128
_LPAD = 1024
_BLK = 512
_NSTEPS = _B // _BLK


def _seg_body(f_ref, lab_ref, sum_ref, cnt_ref):
    i = pl.program_id(0)

    @pl.when(i == 0)
    def _init():
        sum_ref[...] = jnp.zeros_like(sum_ref)
        cnt_ref[...] = jnp.zeros_like(cnt_ref)

    lab = lab_ref[...].reshape(1, _BLK)
    rowid = lax.broadcasted_iota(jnp.int32, (_LPAD, _BLK), 0)
    onehot = jnp.where(rowid == lab, 1.0, 0.0).astype(jnp.float32)
    f = f_ref[...]
    sum_ref[0] += lax.dot_general(
        onehot, f, (((1,), (0,)), ((), ())),
        preferred_element_type=jnp.float32,
        precision=lax.Precision.HIGHEST)
    cnt_ref[0] += jnp.sum(onehot, axis=1, keepdims=True)


def _ce_body(f_ref, lab_ref, psum_ref, pcnt_ref, tau_ref, out_ref, pn_ref):
    i = pl.program_id(0)

    @pl.when(i == 0)
    def _init():
        s = psum_ref[0] + psum_ref[1]
        c = pcnt_ref[0] + pcnt_ref[1]
        mean = s / (c + 1e-6)
        mean = jnp.where(c < 0.5, jnp.zeros_like(mean), mean)
        nrm = jnp.sqrt(jnp.sum(mean * mean, axis=1, keepdims=True))
        pn_ref[...] = mean / jnp.maximum(nrm, 1e-6)
        out_ref[...] = jnp.zeros_like(out_ref)

    f = f_ref[...]
    nrm = jnp.sqrt(jnp.sum(f * f, axis=1, keepdims=True))
    fn = f / jnp.maximum(nrm, 1e-6)
    scale = jnp.exp(jnp.clip(tau_ref[...], 0.0, _CLAMP))
    logits = lax.dot_general(
        fn, pn_ref[...], (((1,), (1,)), ((), ())),
        preferred_element_type=jnp.float32,
        precision=lax.Precision.HIGHEST) * scale
    colid = lax.broadcasted_iota(jnp.int32, (_BLK, _LPAD), 1)
    logits = jnp.where(colid < _NUM_LABELS, logits, jnp.float32(-1e30))
    m = jnp.max(logits, axis=1, keepdims=True)
    ez = jnp.sum(jnp.exp(logits - m), axis=1, keepdims=True)
    logz = jnp.log(ez) + m
    lab = lab_ref[...]
    picked = jnp.sum(jnp.where(colid == lab, logits, 0.0), axis=1, keepdims=True)
    out_ref[...] += jnp.sum(logz - picked)

    @pl.when(i == _NSTEPS - 1)
    def _fin():
        out_ref[...] = out_ref[...] * jnp.float32(1.0 / _B)


def _segment_sum(f_emb, label):
    lab3 = label.reshape(_NSTEPS, 1, _BLK)
    return pl.pallas_call(
        _seg_body,
        grid=(_NSTEPS,),
        in_specs=[
            pl.BlockSpec((_BLK, _D), lambda i: (i, 0)),
            pl.BlockSpec((1, 1, _BLK), lambda i: (i, 0, 0)),
        ],
        out_specs=[
            pl.BlockSpec((2, _LPAD, _D), lambda i: (0, 0, 0)),
            pl.BlockSpec((2, _LPAD, 1), lambda i: (0, 0, 0)),
        ],
        out_shape=[
            jax.ShapeDtypeStruct((2, _LPAD, _D), jnp.float32),
            jax.ShapeDtypeStruct((2, _LPAD, 1), jnp.float32),
        ],
        compiler_params=pltpu.CompilerParams(
            dimension_semantics=("arbitrary",)),
    )(f_emb, lab3)


def _ce_loss(f_emb, label, psum, pcnt, tau):
    labc = label.reshape(_B, 1)
    tau2 = tau.reshape(1, 1)
    acc = pl.pallas_call(
        _ce_body,
        grid=(_NSTEPS,),
        in_specs=[
            pl.BlockSpec((_BLK, _D), lambda i: (i, 0)),
            pl.BlockSpec((_BLK, 1), lambda i: (i, 0)),
            pl.BlockSpec((2, _LPAD, _D), lambda i: (0, 0, 0)),
            pl.BlockSpec((2, _LPAD, 1), lambda i: (0, 0, 0)),
            pl.BlockSpec((1, 1), lambda i: (0, 0)),
        ],
        out_specs=pl.BlockSpec((1, 1), lambda i: (0, 0)),
        out_shape=jax.ShapeDtypeStruct((1, 1), jnp.float32),
        scratch_shapes=[pltpu.VMEM((_LPAD, _D), jnp.float32)],
        compiler_params=pltpu.CompilerParams(
            dimension_semantics=("arbitrary",)),
    )(f_emb, labc, psum, pcnt, tau2)
    return acc[0, 0]


def kernel(f_emb, label, tau):
    psum, pcnt = _segment_sum(f_emb, label)
    return _ce_loss(f_emb, label, psum, pcnt, tau)


# speedup vs baseline: 2.7910x; 2.0366x over previous
import functools

import jax
import jax.numpy as jnp
from jax import lax
from jax.experimental import pallas as pl
from jax.experimental.pallas import tpu as pltpu

_NUM_LABELS = 1000
_CLAMP = 4.6051
_B = 16384
_D = 128
_LPAD = 1024
_BLK = 512
_NSTEPS = _B // _BLK


def _seg_body(f_ref, lab_ref, sum_ref, cnt_ref):
    i = pl.program_id(0)

    @pl.when(i == 0)
    def _init():
        sum_ref[...] = jnp.zeros_like(sum_ref)
        cnt_ref[...] = jnp.zeros_like(cnt_ref)

    lab = lab_ref[...].reshape(1, _BLK)
    rowid = lax.broadcasted_iota(jnp.int32, (_LPAD, _BLK), 0)
    onehot = jnp.where(rowid == lab, 1.0, 0.0).astype(jnp.float32)
    f = f_ref[...]
    sum_ref[0] += lax.dot_general(
        onehot, f, (((1,), (0,)), ((), ())),
        preferred_element_type=jnp.float32,
        precision=lax.Precision.DEFAULT)
    cnt_ref[0] += jnp.sum(onehot, axis=1, keepdims=True)


def _ce_body(f_ref, lab_ref, psum_ref, pcnt_ref, tau_ref, out_ref, pn_ref):
    i = pl.program_id(0)

    @pl.when(i == 0)
    def _init():
        s = psum_ref[0] + psum_ref[1]
        c = pcnt_ref[0] + pcnt_ref[1]
        mean = s / (c + 1e-6)
        mean = jnp.where(c < 0.5, jnp.zeros_like(mean), mean)
        nrm = jnp.sqrt(jnp.sum(mean * mean, axis=1, keepdims=True))
        pn_ref[...] = mean / jnp.maximum(nrm, 1e-6)
        out_ref[...] = jnp.zeros_like(out_ref)

    f = f_ref[...]
    nrm = jnp.sqrt(jnp.sum(f * f, axis=1, keepdims=True))
    fn = f / jnp.maximum(nrm, 1e-6)
    scale = jnp.exp(jnp.clip(tau_ref[...], 0.0, _CLAMP))
    logits = lax.dot_general(
        fn, pn_ref[...], (((1,), (1,)), ((), ())),
        preferred_element_type=jnp.float32,
        precision=lax.Precision.DEFAULT) * scale
    colid = lax.broadcasted_iota(jnp.int32, (_BLK, _LPAD), 1)
    logits = jnp.where(colid < _NUM_LABELS, logits, jnp.float32(-1e30))
    m = jnp.max(logits, axis=1, keepdims=True)
    ez = jnp.sum(jnp.exp(logits - m), axis=1, keepdims=True)
    logz = jnp.log(ez) + m
    lab = lab_ref[...]
    picked = jnp.sum(jnp.where(colid == lab, logits, 0.0), axis=1, keepdims=True)
    out_ref[...] += jnp.sum(logz - picked)

    @pl.when(i == _NSTEPS - 1)
    def _fin():
        out_ref[...] = out_ref[...] * jnp.float32(1.0 / _B)


def _segment_sum(f_emb, label):
    lab3 = label.reshape(_NSTEPS, 1, _BLK)
    return pl.pallas_call(
        _seg_body,
        grid=(_NSTEPS,),
        in_specs=[
            pl.BlockSpec((_BLK, _D), lambda i: (i, 0)),
            pl.BlockSpec((1, 1, _BLK), lambda i: (i, 0, 0)),
        ],
        out_specs=[
            pl.BlockSpec((2, _LPAD, _D), lambda i: (0, 0, 0)),
            pl.BlockSpec((2, _LPAD, 1), lambda i: (0, 0, 0)),
        ],
        out_shape=[
            jax.ShapeDtypeStruct((2, _LPAD, _D), jnp.float32),
            jax.ShapeDtypeStruct((2, _LPAD, 1), jnp.float32),
        ],
        compiler_params=pltpu.CompilerParams(
            dimension_semantics=("arbitrary",)),
    )(f_emb, lab3)


def _ce_loss(f_emb, label, psum, pcnt, tau):
    labc = label.reshape(_B, 1)
    tau2 = tau.reshape(1, 1)
    acc = pl.pallas_call(
        _ce_body,
        grid=(_NSTEPS,),
        in_specs=[
            pl.BlockSpec((_BLK, _D), lambda i: (i, 0)),
            pl.BlockSpec((_BLK, 1), lambda i: (i, 0)),
            pl.BlockSpec((2, _LPAD, _D), lambda i: (0, 0, 0)),
            pl.BlockSpec((2, _LPAD, 1), lambda i: (0, 0, 0)),
            pl.BlockSpec((1, 1), lambda i: (0, 0)),
        ],
        out_specs=pl.BlockSpec((1, 1), lambda i: (0, 0)),
        out_shape=jax.ShapeDtypeStruct((1, 1), jnp.float32),
        scratch_shapes=[pltpu.VMEM((_LPAD, _D), jnp.float32)],
        compiler_params=pltpu.CompilerParams(
            dimension_semantics=("arbitrary",)),
    )(f_emb, labc, psum, pcnt, tau2)
    return acc[0, 0]


def kernel(f_emb, label, tau):
    psum, pcnt = _segment_sum(f_emb, label)
    return _ce_loss(f_emb, label, psum, pcnt, tau)
